# baseline (speedup 1.0000x reference)
import jax
import jax.numpy as jnp
from jax.experimental import pallas as pl

B = 1024
N_PATCHES = 128
D_MODEL = 768
ROW_BLOCK = 2048


def _body(rp_ref, w_ref, s_ref, out_ref):
    out_ref[...] = rp_ref[...] * (w_ref[...] * s_ref[0, 0])


def kernel(n_patches, relative_positions, W, scale):
    rows = B * N_PATCHES
    rp2 = relative_positions.reshape(rows, 1)
    w2 = W.reshape(1, D_MODEL)
    s2 = scale.reshape(1, 1)
    grid = (rows // ROW_BLOCK,)
    out = pl.pallas_call(
        _body,
        grid=grid,
        in_specs=[
            pl.BlockSpec((ROW_BLOCK, 1), lambda i: (i, 0)),
            pl.BlockSpec((1, D_MODEL), lambda i: (0, 0)),
            pl.BlockSpec((1, 1), lambda i: (0, 0)),
        ],
        out_specs=pl.BlockSpec((ROW_BLOCK, D_MODEL), lambda i: (i, 0)),
        out_shape=jax.ShapeDtypeStruct((rows, D_MODEL), jnp.float32),
    )(rp2, w2, s2)
    return out.reshape(B, N_PATCHES, D_MODEL)
